# baseline (device time: 110898 ns/iter reference)
import jax
import jax.numpy as jnp
from jax import lax
from jax.experimental import pallas as pl
from jax.experimental.pallas import tpu as pltpu

N_DEV = 8
N_LAYERS = 3


def kernel(x, Win0, Wout0, Win1, Wout1, Win2, Wout2):
    b, d = x.shape
    chunk = b // N_DEV

    def body(x_ref, win0_ref, wout0_ref, win1_ref, wout1_ref, win2_ref,
             wout2_ref, out_ref, comm_ref, send_sems, recv_sems):
        my = lax.axis_index("i")
        left = lax.rem(my + N_DEV - 1, N_DEV)
        right = lax.rem(my + 1, N_DEV)

        barrier = pltpu.get_barrier_semaphore()
        for nbr in (left, right):
            pl.semaphore_signal(barrier, inc=1, device_id=(nbr,),
                                device_id_type=pl.DeviceIdType.MESH)
        pl.semaphore_wait(barrier, 2)

        wins = (win0_ref, win1_ref, win2_ref)
        wouts = (wout0_ref, wout1_ref, wout2_ref)

        x_val = x_ref[...].astype(jnp.bfloat16)
        for l in range(N_LAYERS):
            h = jnp.dot(x_val, wins[l][...].astype(jnp.bfloat16),
                        preferred_element_type=jnp.float32)
            h = jnp.maximum(h, 0.0).astype(jnp.bfloat16)
            p = jnp.dot(h, wouts[l][...].astype(jnp.bfloat16),
                        preferred_element_type=jnp.float32)
            comm_ref[l, my] = p.astype(jnp.bfloat16)

            o_send = my
            for hop in range(N_DEV - 1):
                o_recv = lax.rem(o_send + N_DEV - 1, N_DEV)
                send = pltpu.make_async_remote_copy(
                    src_ref=comm_ref.at[l, o_send],
                    dst_ref=comm_ref.at[l, o_send],
                    send_sem=send_sems.at[l, hop],
                    recv_sem=recv_sems.at[l, hop],
                    device_id=(right,),
                    device_id_type=pl.DeviceIdType.MESH,
                )
                send.start()
                recv = pltpu.make_async_remote_copy(
                    src_ref=comm_ref.at[l, o_recv],
                    dst_ref=comm_ref.at[l, o_recv],
                    send_sem=send_sems.at[l, hop],
                    recv_sem=recv_sems.at[l, hop],
                    device_id=(left,),
                    device_id_type=pl.DeviceIdType.MESH,
                )
                recv.wait_recv()
                send.wait_send()
                o_send = o_recv

            if l < N_LAYERS - 1:
                acc = comm_ref[l, 0].astype(jnp.float32)
                for s in range(1, N_DEV):
                    acc = acc + comm_ref[l, s].astype(jnp.float32)
                x_val = acc.astype(jnp.bfloat16)
            else:
                acc = comm_ref[l, 0, pl.ds(my * chunk, chunk), :].astype(
                    jnp.float32)
                for s in range(1, N_DEV):
                    acc = acc + comm_ref[
                        l, s, pl.ds(my * chunk, chunk), :].astype(jnp.float32)
                out_ref[...] = acc

    return pl.pallas_call(
        body,
        out_shape=jax.ShapeDtypeStruct((chunk, d), jnp.float32),
        in_specs=[pl.BlockSpec(memory_space=pltpu.VMEM)] * 7,
        out_specs=pl.BlockSpec(memory_space=pltpu.VMEM),
        scratch_shapes=[
            pltpu.VMEM((N_LAYERS, N_DEV, b, d), jnp.bfloat16),
            pltpu.SemaphoreType.DMA((N_LAYERS, N_DEV - 1)),
            pltpu.SemaphoreType.DMA((N_LAYERS, N_DEV - 1)),
        ],
        compiler_params=pltpu.CompilerParams(collective_id=0),
    )(x, Win0, Wout0, Win1, Wout1, Win2, Wout2)


# device time: 33081 ns/iter; 3.3523x vs baseline; 3.3523x over previous
import jax
import jax.numpy as jnp
from jax import lax
from jax.experimental import pallas as pl
from jax.experimental.pallas import tpu as pltpu

N_DEV = 8
N_LAYERS = 3


def kernel(x, Win0, Wout0, Win1, Wout1, Win2, Wout2):
    b, d = x.shape
    chunk = b // N_DEV

    def body(x_ref, win0_ref, wout0_ref, win1_ref, wout1_ref, win2_ref,
             wout2_ref, out_ref, pbuf, rbuf, gbuf,
             red_send_sems, red_recv_sems, gat_send_sems, gat_recv_sems):
        my = lax.axis_index("i")

        barrier = pltpu.get_barrier_semaphore()
        for j in range(1, N_DEV):
            peer = lax.rem(my + j, N_DEV)
            pl.semaphore_signal(barrier, inc=1, device_id=(peer,),
                                device_id_type=pl.DeviceIdType.MESH)
        pl.semaphore_wait(barrier, N_DEV - 1)

        wins = (win0_ref, win1_ref, win2_ref)
        wouts = (wout0_ref, wout1_ref, wout2_ref)

        x_val = x_ref[...].astype(jnp.bfloat16)
        for l in range(N_LAYERS):
            h = jnp.dot(x_val, wins[l][...].astype(jnp.bfloat16),
                        preferred_element_type=jnp.float32)
            h = jnp.maximum(h, 0.0).astype(jnp.bfloat16)
            p = jnp.dot(h, wouts[l][...].astype(jnp.bfloat16),
                        preferred_element_type=jnp.float32)
            pbuf[l] = p.astype(jnp.bfloat16)

            sends = []
            for j in range(1, N_DEV):
                t = lax.rem(my + j, N_DEV)
                s = pltpu.make_async_remote_copy(
                    src_ref=pbuf.at[l, pl.ds(t * chunk, chunk), :],
                    dst_ref=rbuf.at[l, my],
                    send_sem=red_send_sems.at[l],
                    recv_sem=red_recv_sems.at[l],
                    device_id=(t,),
                    device_id_type=pl.DeviceIdType.MESH,
                )
                s.start()
                sends.append(s)
            rbuf[l, my] = pbuf[l, pl.ds(my * chunk, chunk), :]
            for j in range(1, N_DEV):
                src = lax.rem(my + N_DEV - j, N_DEV)
                r = pltpu.make_async_remote_copy(
                    src_ref=rbuf.at[l, src],
                    dst_ref=rbuf.at[l, src],
                    send_sem=red_send_sems.at[l],
                    recv_sem=red_recv_sems.at[l],
                    device_id=(src,),
                    device_id_type=pl.DeviceIdType.MESH,
                )
                r.wait_recv()
            for s in sends:
                s.wait_send()

            acc = rbuf[l, 0].astype(jnp.float32)
            for s_i in range(1, N_DEV):
                acc = acc + rbuf[l, s_i].astype(jnp.float32)

            if l == N_LAYERS - 1:
                out_ref[...] = acc
            else:
                gbuf[l, pl.ds(my * chunk, chunk), :] = acc.astype(jnp.bfloat16)
                sends = []
                for j in range(1, N_DEV):
                    t = lax.rem(my + j, N_DEV)
                    s = pltpu.make_async_remote_copy(
                        src_ref=gbuf.at[l, pl.ds(my * chunk, chunk), :],
                        dst_ref=gbuf.at[l, pl.ds(my * chunk, chunk), :],
                        send_sem=gat_send_sems.at[l],
                        recv_sem=gat_recv_sems.at[l],
                        device_id=(t,),
                        device_id_type=pl.DeviceIdType.MESH,
                    )
                    s.start()
                    sends.append(s)
                for j in range(1, N_DEV):
                    src = lax.rem(my + N_DEV - j, N_DEV)
                    r = pltpu.make_async_remote_copy(
                        src_ref=gbuf.at[l, pl.ds(src * chunk, chunk), :],
                        dst_ref=gbuf.at[l, pl.ds(src * chunk, chunk), :],
                        send_sem=gat_send_sems.at[l],
                        recv_sem=gat_recv_sems.at[l],
                        device_id=(src,),
                        device_id_type=pl.DeviceIdType.MESH,
                    )
                    r.wait_recv()
                for s in sends:
                    s.wait_send()
                x_val = gbuf[l]

    return pl.pallas_call(
        body,
        out_shape=jax.ShapeDtypeStruct((chunk, d), jnp.float32),
        in_specs=[pl.BlockSpec(memory_space=pltpu.VMEM)] * 7,
        out_specs=pl.BlockSpec(memory_space=pltpu.VMEM),
        scratch_shapes=[
            pltpu.VMEM((N_LAYERS, b, d), jnp.bfloat16),
            pltpu.VMEM((N_LAYERS, N_DEV, chunk, d), jnp.bfloat16),
            pltpu.VMEM((N_LAYERS, b, d), jnp.bfloat16),
            pltpu.SemaphoreType.DMA((N_LAYERS,)),
            pltpu.SemaphoreType.DMA((N_LAYERS,)),
            pltpu.SemaphoreType.DMA((N_LAYERS,)),
            pltpu.SemaphoreType.DMA((N_LAYERS,)),
        ],
        compiler_params=pltpu.CompilerParams(collective_id=0),
    )(x, Win0, Wout0, Win1, Wout1, Win2, Wout2)
